# baseline (speedup 1.0000x reference)
import functools
import jax
import jax.numpy as jnp
from jax import lax
from jax.experimental import pallas as pl
from jax.experimental.pallas import tpu as pltpu
from jax.experimental.pallas import tpu_sc as plsc
from jax.experimental.compute_on import compute_on

_NC = 2
_NS = 16
_LANES = 16
_CHR = 16


def _tc_body(x_ref, pe_ref, o_ref):
    o_ref[...] = x_ref[...] + pe_ref[...][None, :, :]


def _tc_add(x, pe, nb, bs=512):
    B, L, D = x.shape
    return pl.pallas_call(
        _tc_body,
        grid=(L // bs, nb),
        in_specs=[
            pl.BlockSpec((1, bs, D), lambda i, b: (b, i, 0)),
            pl.BlockSpec((bs, D), lambda i, b: (i, 0)),
        ],
        out_specs=pl.BlockSpec((1, bs, D), lambda i, b: (b, i, 0)),
        out_shape=jax.ShapeDtypeStruct((B, L, D), x.dtype),
    )(x, pe)


def _sc_add(xf, per, row_lo):
    R, D = per.shape
    NW = _NC * _NS
    seg = R // NW
    n_chunks = seg // _CHR
    n_half = n_chunks // 2
    mesh = plsc.VectorSubcoreMesh(core_axis_name="c", subcore_axis_name="s")

    @functools.partial(
        pl.kernel,
        out_type=jax.ShapeDtypeStruct((R, D), jnp.float32),
        mesh=mesh,
        scratch_types=[
            pltpu.VMEM((2, _CHR, D), jnp.float32),
            pltpu.VMEM((2, _CHR, D), jnp.float32),
            pltpu.VMEM((2, _CHR, D), jnp.float32),
            pltpu.SemaphoreType.DMA,
            pltpu.SemaphoreType.DMA,
            pltpu.SemaphoreType.DMA,
            pltpu.SemaphoreType.DMA,
            pltpu.SemaphoreType.DMA,
            pltpu.SemaphoreType.DMA,
        ],
    )
    def body(x_hbm, pe_hbm, out_hbm, xbuf, pebuf, obuf,
             xs0, xs1, ps0, ps1, os0, os1):
        wid = lax.axis_index("s") * _NC + lax.axis_index("c")
        r0 = wid * seg
        xsem = (xs0, xs1)
        psem = (ps0, ps1)
        osem = (os0, os1)

        def load_descs(ci, par):
            row = r0 + ci * _CHR
            return [
                pltpu.make_async_copy(x_hbm.at[pl.ds(row_lo + row, _CHR)],
                                      xbuf.at[par], xsem[par]),
                pltpu.make_async_copy(pe_hbm.at[pl.ds(row, _CHR)],
                                      pebuf.at[par], psem[par]),
            ]

        def store_desc(ci, par):
            return pltpu.make_async_copy(
                obuf.at[par], out_hbm.at[pl.ds(r0 + ci * _CHR, _CHR)],
                osem[par])

        for par in range(2):
            for cp in load_descs(par, par):
                cp.start()

        def loop_body(h, _):
            for par in range(2):
                ci = 2 * h + par
                for cp in load_descs(ci, par):
                    cp.wait()

                @pl.when(h >= 1)
                def _():
                    store_desc(ci - 2, par).wait()

                @plsc.parallel_loop(0, _CHR)
                def _(i):
                    for j in range(D // _LANES):
                        s = pl.ds(j * _LANES, _LANES)
                        obuf[par, i, s] = xbuf[par, i, s] + pebuf[par, i, s]

                store_desc(ci, par).start()

                @pl.when(h < n_half - 1)
                def _():
                    for cp in load_descs(ci + 2, par):
                        cp.start()
            return 0

        lax.fori_loop(0, n_half, loop_body, 0)
        for par in range(2):
            store_desc(n_chunks - 2 + par, par).wait()

    return body(xf, per)


def kernel(x, pe):
    B, L, D = x.shape
    pef = pe[:L]
    sc_out = _sc_add(x.reshape(B * L, D), pef, (B - 1) * L)
    tc_out = _tc_add(x, pef, B - 1)
    return jnp.concatenate([tc_out[: B - 1], sc_out[None]], axis=0)


# speedup vs baseline: 3.0062x; 3.0062x over previous
import jax
import jax.numpy as jnp
from jax.experimental import pallas as pl


def _add_body(x_ref, pe_ref, o_ref):
    o_ref[...] = x_ref[...] + pe_ref[...][None, :, :]


def kernel(x, pe):
    B, L, D = x.shape
    bs = 512
    return pl.pallas_call(
        _add_body,
        grid=(L // bs,),
        in_specs=[
            pl.BlockSpec((B, bs, D), lambda i: (0, i, 0)),
            pl.BlockSpec((bs, D), lambda i: (i, 0)),
        ],
        out_specs=pl.BlockSpec((B, bs, D), lambda i: (0, i, 0)),
        out_shape=jax.ShapeDtypeStruct((B, L, D), x.dtype),
    )(x, pe[:L])
